# Initial kernel scaffold; baseline (speedup 1.0000x reference)
import jax
import jax.numpy as jnp
from jax.experimental import pallas as pl


def kernel(source_stack, target_stack, W1, b1, W2, b2, w, b, task_vec):
    raise NotImplementedError("write your pallas kernel here")


# speedup vs baseline: 25.0897x; 25.0897x over previous
import functools

import jax
import jax.numpy as jnp
from jax.experimental import pallas as pl
from jax.experimental.pallas import tpu as pltpu

_S_NUM = 2048
_T_NUM = 512
_N_TOT = _S_NUM + _T_NUM
_K_LOW = 943717
_MAX_FINITE_BITS = 0x7F7FFFFF


def _body(s_ref, t_ref, w1_ref, b1_ref, w2_ref, b2_ref, w_ref, bias_ref,
          task_ref, out_ref):
    f32 = jnp.float32
    S = s_ref[...]
    T = t_ref[...]

    dot = functools.partial(jax.lax.dot_general,
                            preferred_element_type=jnp.float32)

    scores = jnp.maximum(
        dot(S, T, (((1,), (1,)), ((), ()))), 0.0)
    bits = jax.lax.bitcast_convert_type(scores, jnp.int32)

    k_low = jnp.int32(_K_LOW)

    def bs_step(_, lohi):
        lo, hi = lohi
        mid = lo + (hi - lo) // 2
        cnt = jnp.sum((bits <= mid).astype(jnp.int32))
        ge = cnt >= k_low + 1
        lo = jnp.where(ge, lo, mid + 1)
        hi = jnp.where(ge, mid, hi)
        return lo, hi

    lo0 = jnp.int32(0)
    hi0 = jnp.int32(_MAX_FINITE_BITS)
    _, vk_bits = jax.lax.fori_loop(0, 31, bs_step, (lo0, hi0))

    cnt_le = jnp.sum((bits <= vk_bits).astype(jnp.int32))
    nxt_bits = jnp.min(jnp.where(bits > vk_bits, bits,
                                 jnp.int32(_MAX_FINITE_BITS)))
    vk1_bits = jnp.where(cnt_le >= k_low + 2, vk_bits, nxt_bits)

    vk = jax.lax.bitcast_convert_type(vk_bits, f32)
    vk1 = jax.lax.bitcast_convert_type(vk1_bits, f32)
    md = vk + (vk1 - vk) * f32(0.5)

    mask = (scores > md).astype(f32)
    ones_t = jnp.ones((_T_NUM, 1), f32)
    ones_s = jnp.ones((_S_NUM, 1), f32)
    deg_s = dot(mask, ones_t, (((1,), (0,)), ((), ()))) + 1.0
    deg_t = dot(mask, ones_s, (((0,), (0,)), ((), ()))) + 1.0
    dinv_s = jax.lax.rsqrt(deg_s)
    dinv_t = jax.lax.rsqrt(deg_t)

    W1 = w1_ref[...]
    b1 = b1_ref[...]
    W2 = w2_ref[...]
    b2 = b2_ref[...]

    def agg(hs, ht):
        ms = dot(mask, dinv_t * ht, (((1,), (0,)), ((), ())))
        mt = dot(mask, dinv_s * hs, (((0,), (0,)), ((), ())))
        out_s = dinv_s * (dinv_s * hs + ms)
        out_t = dinv_t * (dinv_t * ht + mt)
        return out_s, out_t

    hs1 = dot(S, W1, (((1,), (0,)), ((), ())))
    ht1 = dot(T, W1, (((1,), (0,)), ((), ())))
    as1, at1 = agg(hs1, ht1)
    h1s = jnp.maximum(as1 + b1, 0.0)
    h1t = jnp.maximum(at1 + b1, 0.0)

    hs2 = dot(h1s, W2, (((1,), (0,)), ((), ())))
    ht2 = dot(h1t, W2, (((1,), (0,)), ((), ())))
    emb_s, emb_t = agg(hs2, ht2)
    emb_s = emb_s + b2
    emb_t = emb_t + b2

    tgt = jnp.sum(emb_t, axis=0, keepdims=True) * f32(1.0 / _T_NUM)
    wv = (w_ref[...] * tgt.T)
    soutar = dot(emb_s, wv, (((1,), (0,)), ((), ()))) + bias_ref[...]
    out = 0.5 * jax.nn.sigmoid(soutar) + 0.5 * jax.nn.sigmoid(task_ref[...])
    out_ref[...] = out


@jax.jit
def kernel(source_stack, target_stack, W1, b1, W2, b2, w, b, task_vec):
    out = pl.pallas_call(
        _body,
        out_shape=jax.ShapeDtypeStruct((_S_NUM, 1), jnp.float32),
    )(source_stack, target_stack, W1, b1.reshape(1, -1), W2,
      b2.reshape(1, -1), w, b.reshape(1, 1), task_vec)
    return out


# speedup vs baseline: 25.4026x; 1.0125x over previous
import functools

import jax
import jax.numpy as jnp
from jax.experimental import pallas as pl
from jax.experimental.pallas import tpu as pltpu

_S_NUM = 2048
_T_NUM = 512
_N_TOT = _S_NUM + _T_NUM
_K_LOW = 943717
_MAX_FINITE_BITS = 0x7F7FFFFF


def _body(s_ref, t_ref, w1_ref, b1_ref, w2_ref, b2_ref, w_ref, bias_ref,
          task_ref, out_ref):
    f32 = jnp.float32
    S = s_ref[...]
    T = t_ref[...]

    dot = functools.partial(jax.lax.dot_general,
                            preferred_element_type=jnp.float32)

    scores = jnp.maximum(
        dot(S, T, (((1,), (1,)), ((), ()))), 0.0)

    k_low = jnp.int32(_K_LOW)

    def bs_step(_, lohi):
        lo, hi = lohi
        mid = lo + (hi - lo) // 2
        t = jax.lax.bitcast_convert_type(mid, f32)
        cnt = jnp.sum(jnp.where(scores <= t, f32(1.0), f32(0.0)))
        ge = cnt >= f32(_K_LOW + 1)
        lo = jnp.where(ge, lo, mid + 1)
        hi = jnp.where(ge, mid, hi)
        return lo, hi

    lo0 = jnp.int32(0)
    hi0 = jnp.int32(_MAX_FINITE_BITS)
    _, vk_bits = jax.lax.fori_loop(0, 31, bs_step, (lo0, hi0))

    vk = jax.lax.bitcast_convert_type(vk_bits, f32)
    cnt_le = jnp.sum(jnp.where(scores <= vk, f32(1.0), f32(0.0)))
    big = jax.lax.bitcast_convert_type(jnp.int32(_MAX_FINITE_BITS), f32)
    vk1_cand = jnp.min(jnp.where(scores > vk, scores, big))
    vk1 = jnp.where(cnt_le >= f32(_K_LOW + 2), vk, vk1_cand)
    md = vk + (vk1 - vk) * f32(0.5)

    mask = (scores > md).astype(f32)
    ones_t = jnp.ones((_T_NUM, 1), f32)
    ones_s = jnp.ones((_S_NUM, 1), f32)
    deg_s = dot(mask, ones_t, (((1,), (0,)), ((), ()))) + 1.0
    deg_t = dot(mask, ones_s, (((0,), (0,)), ((), ()))) + 1.0
    dinv_s = jax.lax.rsqrt(deg_s)
    dinv_t = jax.lax.rsqrt(deg_t)

    W1 = w1_ref[...]
    b1 = b1_ref[...]
    W2 = w2_ref[...]
    b2 = b2_ref[...]

    def agg(hs, ht):
        ms = dot(mask, dinv_t * ht, (((1,), (0,)), ((), ())))
        mt = dot(mask, dinv_s * hs, (((0,), (0,)), ((), ())))
        out_s = dinv_s * (dinv_s * hs + ms)
        out_t = dinv_t * (dinv_t * ht + mt)
        return out_s, out_t

    hs1 = dot(S, W1, (((1,), (0,)), ((), ())))
    ht1 = dot(T, W1, (((1,), (0,)), ((), ())))
    as1, at1 = agg(hs1, ht1)
    h1s = jnp.maximum(as1 + b1, 0.0)
    h1t = jnp.maximum(at1 + b1, 0.0)

    hs2 = dot(h1s, W2, (((1,), (0,)), ((), ())))
    ht2 = dot(h1t, W2, (((1,), (0,)), ((), ())))
    emb_s, emb_t = agg(hs2, ht2)
    emb_s = emb_s + b2
    emb_t = emb_t + b2

    tgt = jnp.sum(emb_t, axis=0, keepdims=True) * f32(1.0 / _T_NUM)
    wv = (w_ref[...] * tgt.T)
    soutar = dot(emb_s, wv, (((1,), (0,)), ((), ()))) + bias_ref[...]
    out = 0.5 * jax.nn.sigmoid(soutar) + 0.5 * jax.nn.sigmoid(task_ref[...])
    out_ref[...] = out


@jax.jit
def kernel(source_stack, target_stack, W1, b1, W2, b2, w, b, task_vec):
    out = pl.pallas_call(
        _body,
        out_shape=jax.ShapeDtypeStruct((_S_NUM, 1), jnp.float32),
    )(source_stack, target_stack, W1, b1.reshape(1, -1), W2,
      b2.reshape(1, -1), w, b.reshape(1, 1), task_vec)
    return out


# speedup vs baseline: 25.4064x; 1.0002x over previous
import functools

import jax
import jax.numpy as jnp
from jax.experimental import pallas as pl
from jax.experimental.pallas import tpu as pltpu

_S_NUM = 2048
_T_NUM = 512
_N_TOT = _S_NUM + _T_NUM
_K_LOW = 943717
_MAX_FINITE_BITS = 0x7F7FFFFF


def _body(s_ref, t_ref, w1_ref, b1_ref, w2_ref, b2_ref, w_ref, bias_ref,
          task_ref, out_ref):
    f32 = jnp.float32
    S = s_ref[...]
    T = t_ref[...]

    dot = functools.partial(jax.lax.dot_general,
                            preferred_element_type=jnp.float32)

    scores = jnp.maximum(
        dot(S, T, (((1,), (1,)), ((), ()))), 0.0)

    k_low = jnp.int32(_K_LOW)

    def bs_step(_, lohi):
        lo, hi = lohi
        mid = lo + (hi - lo) // 2
        t = jax.lax.bitcast_convert_type(mid, f32)
        cnt = jnp.count_nonzero(scores <= t)
        ge = cnt >= k_low + 1
        lo = jnp.where(ge, lo, mid + 1)
        hi = jnp.where(ge, mid, hi)
        return lo, hi

    lo0 = jnp.int32(0)
    hi0 = jnp.int32(_MAX_FINITE_BITS)
    _, vk_bits = jax.lax.fori_loop(0, 31, bs_step, (lo0, hi0))

    vk = jax.lax.bitcast_convert_type(vk_bits, f32)
    cnt_le = jnp.sum(jnp.where(scores <= vk, f32(1.0), f32(0.0)))
    big = jax.lax.bitcast_convert_type(jnp.int32(_MAX_FINITE_BITS), f32)
    vk1_cand = jnp.min(jnp.where(scores > vk, scores, big))
    vk1 = jnp.where(cnt_le >= f32(_K_LOW + 2), vk, vk1_cand)
    md = vk + (vk1 - vk) * f32(0.5)

    mask = (scores > md).astype(f32)
    ones_t = jnp.ones((_T_NUM, 1), f32)
    ones_s = jnp.ones((_S_NUM, 1), f32)
    deg_s = dot(mask, ones_t, (((1,), (0,)), ((), ()))) + 1.0
    deg_t = dot(mask, ones_s, (((0,), (0,)), ((), ()))) + 1.0
    dinv_s = jax.lax.rsqrt(deg_s)
    dinv_t = jax.lax.rsqrt(deg_t)

    W1 = w1_ref[...]
    b1 = b1_ref[...]
    W2 = w2_ref[...]
    b2 = b2_ref[...]

    def agg(hs, ht):
        ms = dot(mask, dinv_t * ht, (((1,), (0,)), ((), ())))
        mt = dot(mask, dinv_s * hs, (((0,), (0,)), ((), ())))
        out_s = dinv_s * (dinv_s * hs + ms)
        out_t = dinv_t * (dinv_t * ht + mt)
        return out_s, out_t

    hs1 = dot(S, W1, (((1,), (0,)), ((), ())))
    ht1 = dot(T, W1, (((1,), (0,)), ((), ())))
    as1, at1 = agg(hs1, ht1)
    h1s = jnp.maximum(as1 + b1, 0.0)
    h1t = jnp.maximum(at1 + b1, 0.0)

    hs2 = dot(h1s, W2, (((1,), (0,)), ((), ())))
    ht2 = dot(h1t, W2, (((1,), (0,)), ((), ())))
    emb_s, emb_t = agg(hs2, ht2)
    emb_s = emb_s + b2
    emb_t = emb_t + b2

    tgt = jnp.sum(emb_t, axis=0, keepdims=True) * f32(1.0 / _T_NUM)
    wv = (w_ref[...] * tgt.T)
    soutar = dot(emb_s, wv, (((1,), (0,)), ((), ()))) + bias_ref[...]
    out = 0.5 * jax.nn.sigmoid(soutar) + 0.5 * jax.nn.sigmoid(task_ref[...])
    out_ref[...] = out


@jax.jit
def kernel(source_stack, target_stack, W1, b1, W2, b2, w, b, task_vec):
    out = pl.pallas_call(
        _body,
        out_shape=jax.ShapeDtypeStruct((_S_NUM, 1), jnp.float32),
    )(source_stack, target_stack, W1, b1.reshape(1, -1), W2,
      b2.reshape(1, -1), w, b.reshape(1, 1), task_vec)
    return out
